# baseline (device time: 197635 ns/iter reference)
import jax
import jax.numpy as jnp
from jax import lax
from jax.experimental import pallas as pl
from jax.experimental.pallas import tpu as pltpu

N_DEV = 4
M = 512
K = 1024
NL = 8192
NG = N_DEV * NL
TW = 1024
NT = NL // TW
TN = 2048
MH = M // 2


def _fused(x, W):
    def body(x_ref, w_hbm, dummy_ref, out_ref, comm_ref, wt_ref,
             w_sems, send_sems, recv_sems, s2_send, s2_recv, copy_sem):
        del dummy_ref
        my = lax.axis_index("i")
        left = lax.rem(my + (N_DEV - 1), N_DEV)
        right = lax.rem(my + 1, N_DEV)

        barrier_sem = pltpu.get_barrier_semaphore()
        for nbr in (left, right):
            pl.semaphore_signal(
                barrier_sem, inc=1,
                device_id=(nbr,), device_id_type=pl.DeviceIdType.MESH,
            )
        pl.semaphore_wait(barrier_sem, 2)

        def remote_copy(src, dst, sends, recvs, idx, dst_dev):
            return pltpu.make_async_remote_copy(
                src_ref=src,
                dst_ref=dst,
                send_sem=sends.at[idx],
                recv_sem=recvs.at[idx],
                device_id=(dst_dev,),
                device_id_type=pl.DeviceIdType.MESH,
            )

        def cols(origin):
            return pl.ds(pl.multiple_of(origin * NL, NL), NL)

        def sub_cols(origin, j):
            return pl.ds(pl.multiple_of(origin * NL + j * TW, TW), TW)

        def w_dma(j):
            return pltpu.make_async_copy(
                w_hbm.at[:, pl.ds(j * TW, TW)],
                wt_ref.at[j % 2],
                w_sems.at[j % 2],
            )

        w_dma(0).start()
        xb = x_ref[...].astype(jnp.bfloat16)
        hop1 = []
        for j in range(NT):
            if j + 1 < NT:
                w_dma(j + 1).start()
            w_dma(j).wait()
            wb = wt_ref[j % 2].astype(jnp.bfloat16)
            comm_ref[:, sub_cols(my, j)] = jnp.dot(
                xb, wb, preferred_element_type=jnp.float32
            ).astype(jnp.bfloat16)
            r1j = remote_copy(
                comm_ref.at[:, sub_cols(my, j)],
                comm_ref.at[:, sub_cols(my, j)],
                send_sems, recv_sems, j, right,
            )
            l1j = remote_copy(
                comm_ref.at[:, sub_cols(my, j)],
                comm_ref.at[:, sub_cols(my, j)],
                send_sems, recv_sems, NT + j, left,
            )
            r1j.start()
            l1j.start()
            hop1.append((r1j, l1j))

        def chunk_stats(col0, ms):
            def tbody(t, carry):
                m, s = carry
                l = comm_ref[:, pl.ds(col0 + t * TN, TN)].astype(jnp.float32)
                tm = jnp.max(l, axis=1, keepdims=True)
                te = jnp.exp(l - tm)
                ts = jnp.sum(te, axis=1, keepdims=True)
                nm = jnp.maximum(m, tm)
                return nm, s * jnp.exp(m - nm) + ts * jnp.exp(tm - nm)
            return lax.fori_loop(0, NL // TN, tbody, ms)

        ms = (
            jnp.full((M, 1), -jnp.inf, jnp.float32),
            jnp.zeros((M, 1), jnp.float32),
        )
        ms = chunk_stats(my * NL, ms)

        for r1j, _ in hop1:
            r1j.wait_recv()
        r2 = remote_copy(
            comm_ref.at[pl.ds(0, MH), cols(left)],
            comm_ref.at[pl.ds(0, MH), cols(left)],
            s2_send, s2_recv, 0, right,
        )
        r2.start()
        for _, l1j in hop1:
            l1j.wait_recv()
        l2 = remote_copy(
            comm_ref.at[pl.ds(MH, MH), cols(right)],
            comm_ref.at[pl.ds(MH, MH), cols(right)],
            s2_send, s2_recv, 1, left,
        )
        l2.start()
        for r1j, l1j in hop1:
            r1j.wait_send()
            l1j.wait_send()

        ms = chunk_stats(left * NL, ms)
        ms = chunk_stats(right * NL, ms)

        r2.wait_recv()
        l2.wait_recv()
        r2.wait_send()
        l2.wait_send()
        opp = lax.rem(my + 2, N_DEV)
        ms = chunk_stats(opp * NL, ms)

        m, s = ms
        r = 1.0 / s

        def out_dma(t):
            return pltpu.make_async_copy(
                comm_ref.at[:, pl.ds(t * TN, TN)],
                out_ref.at[:, pl.ds(t * TN, TN)],
                copy_sem,
            )

        def norm_body(t, carry):
            l = comm_ref[:, pl.ds(t * TN, TN)].astype(jnp.float32)
            comm_ref[:, pl.ds(t * TN, TN)] = (
                jnp.exp(l - m) * r
            ).astype(jnp.bfloat16)
            out_dma(t).start()
            return carry

        lax.fori_loop(0, NG // TN, norm_body, jnp.zeros((1, 1), jnp.float32))

        def drain_body(t, carry):
            out_dma(t).wait()
            return carry

        lax.fori_loop(0, NG // TN, drain_body, jnp.zeros((1, 1), jnp.float32))

    return pl.pallas_call(
        body,
        out_shape=jax.ShapeDtypeStruct((M, NG), jnp.bfloat16),
        in_specs=[
            pl.BlockSpec(memory_space=pltpu.VMEM),
            pl.BlockSpec(memory_space=pl.ANY),
            pl.BlockSpec(memory_space=pl.ANY),
        ],
        out_specs=pl.BlockSpec(memory_space=pl.ANY),
        input_output_aliases={2: 0},
        scratch_shapes=[
            pltpu.VMEM((M, NG), jnp.bfloat16),
            pltpu.VMEM((2, K, TW), jnp.float32),
            pltpu.SemaphoreType.DMA((2,)),
            pltpu.SemaphoreType.DMA((2 * NT,)),
            pltpu.SemaphoreType.DMA((2 * NT,)),
            pltpu.SemaphoreType.DMA((2,)),
            pltpu.SemaphoreType.DMA((2,)),
            pltpu.SemaphoreType.DMA,
        ],
        compiler_params=pltpu.CompilerParams(
            collective_id=0, vmem_limit_bytes=60 * 1024 * 1024
        ),
    )(x, W, jnp.zeros((M, NG), jnp.bfloat16))


def kernel(x, W):
    return _fused(x, W)


# device time: 183248 ns/iter; 1.0785x vs baseline; 1.0785x over previous
import jax
import jax.numpy as jnp
from jax import lax
from jax.experimental import pallas as pl
from jax.experimental.pallas import tpu as pltpu

N_DEV = 4
M = 512
K = 1024
NL = 8192
NG = N_DEV * NL
TW = 1024
NT = NL // TW
TN = 2048
MH = M // 2


def _gather_stats(x, W):
    def body(x_ref, w_hbm, lgout_ref, m_ref, r_ref, comm_ref, wt_ref,
             w_sems, send_sems, recv_sems, s2_send, s2_recv, lg_sems):
        my = lax.axis_index("i")
        left = lax.rem(my + (N_DEV - 1), N_DEV)
        right = lax.rem(my + 1, N_DEV)

        barrier_sem = pltpu.get_barrier_semaphore()
        for nbr in (left, right):
            pl.semaphore_signal(
                barrier_sem, inc=1,
                device_id=(nbr,), device_id_type=pl.DeviceIdType.MESH,
            )
        pl.semaphore_wait(barrier_sem, 2)

        def remote_copy(src, dst, sends, recvs, idx, dst_dev):
            return pltpu.make_async_remote_copy(
                src_ref=src,
                dst_ref=dst,
                send_sem=sends.at[idx],
                recv_sem=recvs.at[idx],
                device_id=(dst_dev,),
                device_id_type=pl.DeviceIdType.MESH,
            )

        def cols(origin):
            return pl.ds(pl.multiple_of(origin * NL, NL), NL)

        def sub_cols(origin, j):
            return pl.ds(pl.multiple_of(origin * NL + j * TW, TW), TW)

        def w_dma(j):
            return pltpu.make_async_copy(
                w_hbm.at[:, pl.ds(j * TW, TW)],
                wt_ref.at[j % 2],
                w_sems.at[j % 2],
            )

        def lg_dma(origin, slot):
            return pltpu.make_async_copy(
                comm_ref.at[:, cols(origin)],
                lgout_ref.at[:, cols(origin)],
                lg_sems.at[slot],
            )

        w_dma(0).start()
        xb = x_ref[...].astype(jnp.bfloat16)
        hop1 = []
        for j in range(NT):
            if j + 1 < NT:
                w_dma(j + 1).start()
            w_dma(j).wait()
            wb = wt_ref[j % 2].astype(jnp.bfloat16)
            comm_ref[:, sub_cols(my, j)] = jnp.dot(
                xb, wb, preferred_element_type=jnp.float32
            ).astype(jnp.bfloat16)
            r1j = remote_copy(
                comm_ref.at[:, sub_cols(my, j)],
                comm_ref.at[:, sub_cols(my, j)],
                send_sems, recv_sems, j, right,
            )
            l1j = remote_copy(
                comm_ref.at[:, sub_cols(my, j)],
                comm_ref.at[:, sub_cols(my, j)],
                send_sems, recv_sems, NT + j, left,
            )
            r1j.start()
            l1j.start()
            hop1.append((r1j, l1j))
        lg_dma(my, 0).start()

        def chunk_stats(col0, ms):
            def tbody(t, carry):
                m, s = carry
                l = comm_ref[:, pl.ds(col0 + t * TN, TN)].astype(jnp.float32)
                tm = jnp.max(l, axis=1, keepdims=True)
                te = jnp.exp(l - tm)
                ts = jnp.sum(te, axis=1, keepdims=True)
                nm = jnp.maximum(m, tm)
                return nm, s * jnp.exp(m - nm) + ts * jnp.exp(tm - nm)
            return lax.fori_loop(0, NL // TN, tbody, ms)

        ms = (
            jnp.full((M, 1), -jnp.inf, jnp.float32),
            jnp.zeros((M, 1), jnp.float32),
        )
        ms = chunk_stats(my * NL, ms)

        for r1j, _ in hop1:
            r1j.wait_recv()
        r2 = remote_copy(
            comm_ref.at[pl.ds(0, MH), cols(left)],
            comm_ref.at[pl.ds(0, MH), cols(left)],
            s2_send, s2_recv, 0, right,
        )
        r2.start()
        lg_dma(left, 1).start()
        for _, l1j in hop1:
            l1j.wait_recv()
        l2 = remote_copy(
            comm_ref.at[pl.ds(MH, MH), cols(right)],
            comm_ref.at[pl.ds(MH, MH), cols(right)],
            s2_send, s2_recv, 1, left,
        )
        l2.start()
        lg_dma(right, 2).start()
        for r1j, l1j in hop1:
            r1j.wait_send()
            l1j.wait_send()

        ms = chunk_stats(left * NL, ms)
        ms = chunk_stats(right * NL, ms)

        r2.wait_recv()
        l2.wait_recv()
        r2.wait_send()
        l2.wait_send()
        opp = lax.rem(my + 2, N_DEV)
        lg_dma(opp, 3).start()
        ms = chunk_stats(opp * NL, ms)

        m, s = ms
        m_ref[...] = jnp.broadcast_to(m, (M, 128))
        r_ref[...] = jnp.broadcast_to(1.0 / s, (M, 128))

        for origin, slot in ((my, 0), (left, 1), (right, 2), (opp, 3)):
            lg_dma(origin, slot).wait()

    return pl.pallas_call(
        body,
        out_shape=[
            jax.ShapeDtypeStruct((M, NG), jnp.bfloat16),
            jax.ShapeDtypeStruct((M, 128), jnp.float32),
            jax.ShapeDtypeStruct((M, 128), jnp.float32),
        ],
        in_specs=[
            pl.BlockSpec(memory_space=pltpu.VMEM),
            pl.BlockSpec(memory_space=pl.ANY),
        ],
        out_specs=[
            pl.BlockSpec(memory_space=pl.ANY),
            pl.BlockSpec(memory_space=pltpu.VMEM),
            pl.BlockSpec(memory_space=pltpu.VMEM),
        ],
        scratch_shapes=[
            pltpu.VMEM((M, NG), jnp.bfloat16),
            pltpu.VMEM((2, K, TW), jnp.float32),
            pltpu.SemaphoreType.DMA((2,)),
            pltpu.SemaphoreType.DMA((2 * NT,)),
            pltpu.SemaphoreType.DMA((2 * NT,)),
            pltpu.SemaphoreType.DMA((2,)),
            pltpu.SemaphoreType.DMA((2,)),
            pltpu.SemaphoreType.DMA((4,)),
        ],
        compiler_params=pltpu.CompilerParams(
            collective_id=0, vmem_limit_bytes=60 * 1024 * 1024
        ),
    )(x, W)


def _normalize(lg, m, r):
    def body(lg_ref, m_ref, r_ref, out_ref):
        l = lg_ref[...].astype(jnp.float32)
        out_ref[...] = (
            jnp.exp(l - m_ref[:, 0:1]) * r_ref[:, 0:1]
        ).astype(jnp.bfloat16)

    return pl.pallas_call(
        body,
        grid=(NG // TN,),
        in_specs=[
            pl.BlockSpec((M, TN), lambda t: (0, t)),
            pl.BlockSpec((M, 128), lambda t: (0, 0)),
            pl.BlockSpec((M, 128), lambda t: (0, 0)),
        ],
        out_specs=pl.BlockSpec((M, TN), lambda t: (0, t)),
        out_shape=jax.ShapeDtypeStruct((M, NG), jnp.bfloat16),
    )(lg, m, r)


def kernel(x, W):
    lg, m, r = _gather_stats(x, W)
    return _normalize(lg, m, r)


# device time: 181664 ns/iter; 1.0879x vs baseline; 1.0087x over previous
import jax
import jax.numpy as jnp
from jax import lax
from jax.experimental import pallas as pl
from jax.experimental.pallas import tpu as pltpu

N_DEV = 4
M = 512
K = 1024
NL = 8192
NG = N_DEV * NL
TW = 1024
NT = NL // TW
TN = 2048
MH = M // 2


def _gather_stats(x, W):
    def body(x_ref, w_hbm, lgout_ref, m_ref, r_ref, comm_ref, wt_ref,
             w_sems, send_sems, recv_sems, s2_send, s2_recv, lg_sems):
        my = lax.axis_index("i")
        left = lax.rem(my + (N_DEV - 1), N_DEV)
        right = lax.rem(my + 1, N_DEV)

        barrier_sem = pltpu.get_barrier_semaphore()
        for nbr in (left, right):
            pl.semaphore_signal(
                barrier_sem, inc=1,
                device_id=(nbr,), device_id_type=pl.DeviceIdType.MESH,
            )
        pl.semaphore_wait(barrier_sem, 2)

        def remote_copy(src, dst, sends, recvs, idx, dst_dev):
            return pltpu.make_async_remote_copy(
                src_ref=src,
                dst_ref=dst,
                send_sem=sends.at[idx],
                recv_sem=recvs.at[idx],
                device_id=(dst_dev,),
                device_id_type=pl.DeviceIdType.MESH,
            )

        def cols(origin):
            return pl.ds(pl.multiple_of(origin * NL, NL), NL)

        def sub_cols(origin, j):
            return pl.ds(pl.multiple_of(origin * NL + j * TW, TW), TW)

        def w_dma(j):
            return pltpu.make_async_copy(
                w_hbm.at[:, pl.ds(j * TW, TW)],
                wt_ref.at[j % 2],
                w_sems.at[j % 2],
            )

        def lg_dma(origin, slot):
            return pltpu.make_async_copy(
                comm_ref.at[:, cols(origin)],
                lgout_ref.at[:, cols(origin)],
                lg_sems.at[slot],
            )

        w_dma(0).start()
        xb = x_ref[...].astype(jnp.bfloat16)
        hop1 = []
        for j in range(NT):
            if j + 1 < NT:
                w_dma(j + 1).start()
            w_dma(j).wait()
            wb = wt_ref[j % 2].astype(jnp.bfloat16)
            comm_ref[:, sub_cols(my, j)] = jnp.dot(
                xb, wb, preferred_element_type=jnp.float32
            ).astype(jnp.bfloat16)
            r1j = remote_copy(
                comm_ref.at[:, sub_cols(my, j)],
                comm_ref.at[:, sub_cols(my, j)],
                send_sems, recv_sems, j, right,
            )
            l1j = remote_copy(
                comm_ref.at[:, sub_cols(my, j)],
                comm_ref.at[:, sub_cols(my, j)],
                send_sems, recv_sems, NT + j, left,
            )
            r1j.start()
            l1j.start()
            hop1.append((r1j, l1j))
        lg_dma(my, 0).start()

        def chunk_stats(col0, ms):
            def tbody(t, carry):
                m, s = carry
                l = comm_ref[:, pl.ds(col0 + t * TN, TN)].astype(jnp.float32)
                tm = jnp.max(l, axis=1, keepdims=True)
                te = jnp.exp(l - tm)
                ts = jnp.sum(te, axis=1, keepdims=True)
                nm = jnp.maximum(m, tm)
                return nm, s * jnp.exp(m - nm) + ts * jnp.exp(tm - nm)
            return lax.fori_loop(0, NL // TN, tbody, ms)

        ms = (
            jnp.full((M, 1), -jnp.inf, jnp.float32),
            jnp.zeros((M, 1), jnp.float32),
        )
        ms = chunk_stats(my * NL, ms)

        for r1j, _ in hop1:
            r1j.wait_recv()
        r2 = remote_copy(
            comm_ref.at[pl.ds(0, MH), cols(left)],
            comm_ref.at[pl.ds(0, MH), cols(left)],
            s2_send, s2_recv, 0, right,
        )
        r2.start()
        lg_dma(left, 1).start()
        for _, l1j in hop1:
            l1j.wait_recv()
        l2 = remote_copy(
            comm_ref.at[pl.ds(MH, MH), cols(right)],
            comm_ref.at[pl.ds(MH, MH), cols(right)],
            s2_send, s2_recv, 1, left,
        )
        l2.start()
        lg_dma(right, 2).start()
        for r1j, l1j in hop1:
            r1j.wait_send()
            l1j.wait_send()

        ms = chunk_stats(left * NL, ms)
        ms = chunk_stats(right * NL, ms)

        r2.wait_recv()
        l2.wait_recv()
        r2.wait_send()
        l2.wait_send()
        opp = lax.rem(my + 2, N_DEV)
        lg_dma(opp, 3).start()
        ms = chunk_stats(opp * NL, ms)

        m, s = ms
        m_ref[...] = jnp.broadcast_to(m, (M, 128))
        r_ref[...] = jnp.broadcast_to(1.0 / s, (M, 128))

        for origin, slot in ((my, 0), (left, 1), (right, 2), (opp, 3)):
            lg_dma(origin, slot).wait()

    return pl.pallas_call(
        body,
        out_shape=[
            jax.ShapeDtypeStruct((M, NG), jnp.bfloat16),
            jax.ShapeDtypeStruct((M, 128), jnp.float32),
            jax.ShapeDtypeStruct((M, 128), jnp.float32),
        ],
        in_specs=[
            pl.BlockSpec(memory_space=pltpu.VMEM),
            pl.BlockSpec(memory_space=pl.ANY),
        ],
        out_specs=[
            pl.BlockSpec(memory_space=pl.ANY),
            pl.BlockSpec(memory_space=pltpu.VMEM),
            pl.BlockSpec(memory_space=pltpu.VMEM),
        ],
        scratch_shapes=[
            pltpu.VMEM((M, NG), jnp.bfloat16),
            pltpu.VMEM((2, K, TW), jnp.float32),
            pltpu.SemaphoreType.DMA((2,)),
            pltpu.SemaphoreType.DMA((2 * NT,)),
            pltpu.SemaphoreType.DMA((2 * NT,)),
            pltpu.SemaphoreType.DMA((2,)),
            pltpu.SemaphoreType.DMA((2,)),
            pltpu.SemaphoreType.DMA((4,)),
        ],
        compiler_params=pltpu.CompilerParams(
            collective_id=0, vmem_limit_bytes=60 * 1024 * 1024
        ),
    )(x, W)


def _normalize(lg, m, r):
    def body(lg_ref, m_ref, r_ref, out_ref):
        m = m_ref[:, 0:1].astype(jnp.bfloat16)
        r = r_ref[:, 0:1].astype(jnp.bfloat16)
        out_ref[...] = jnp.exp(lg_ref[...] - m) * r

    return pl.pallas_call(
        body,
        grid=(NG // TN,),
        in_specs=[
            pl.BlockSpec((M, TN), lambda t: (0, t)),
            pl.BlockSpec((M, 128), lambda t: (0, 0)),
            pl.BlockSpec((M, 128), lambda t: (0, 0)),
        ],
        out_specs=pl.BlockSpec((M, TN), lambda t: (0, t)),
        out_shape=jax.ShapeDtypeStruct((M, NG), jnp.bfloat16),
    )(lg, m, r)


def kernel(x, W):
    lg, m, r = _gather_stats(x, W)
    return _normalize(lg, m, r)


# device time: 180803 ns/iter; 1.0931x vs baseline; 1.0048x over previous
import jax
import jax.numpy as jnp
from jax import lax
from jax.experimental import pallas as pl
from jax.experimental.pallas import tpu as pltpu

N_DEV = 4
M = 512
K = 1024
NL = 8192
NG = N_DEV * NL
TW = 1024
NT = NL // TW
TN = 2048
MH = M // 2


def _gather_stats(x, W):
    def body(x_ref, w_hbm, lgout_ref, m_ref, r_ref, comm_ref, wt_ref,
             stats_ref, w_sems, send_sems, recv_sems, s2_send, s2_recv,
             st_send, st_recv, lg_sems):
        my = lax.axis_index("i")
        left = lax.rem(my + (N_DEV - 1), N_DEV)
        right = lax.rem(my + 1, N_DEV)

        barrier_sem = pltpu.get_barrier_semaphore()
        for nbr in (left, right):
            pl.semaphore_signal(
                barrier_sem, inc=1,
                device_id=(nbr,), device_id_type=pl.DeviceIdType.MESH,
            )
        pl.semaphore_wait(barrier_sem, 2)

        def remote_copy(src, dst, sends, recvs, idx, dst_dev):
            return pltpu.make_async_remote_copy(
                src_ref=src,
                dst_ref=dst,
                send_sem=sends.at[idx],
                recv_sem=recvs.at[idx],
                device_id=(dst_dev,),
                device_id_type=pl.DeviceIdType.MESH,
            )

        def cols(origin):
            return pl.ds(pl.multiple_of(origin * NL, NL), NL)

        def sub_cols(origin, j):
            return pl.ds(pl.multiple_of(origin * NL + j * TW, TW), TW)

        def w_dma(j):
            return pltpu.make_async_copy(
                w_hbm.at[:, pl.ds(j * TW, TW)],
                wt_ref.at[j % 2],
                w_sems.at[j % 2],
            )

        def lg_dma(origin, slot):
            return pltpu.make_async_copy(
                comm_ref.at[:, cols(origin)],
                lgout_ref.at[:, cols(origin)],
                lg_sems.at[slot],
            )

        w_dma(0).start()
        xb = x_ref[...].astype(jnp.bfloat16)
        hop1 = []
        m0 = None
        for j in range(NT):
            if j + 1 < NT:
                w_dma(j + 1).start()
            w_dma(j).wait()
            wb = wt_ref[j % 2].astype(jnp.bfloat16)
            lt = jnp.dot(xb, wb, preferred_element_type=jnp.float32)
            comm_ref[:, sub_cols(my, j)] = lt.astype(jnp.bfloat16)
            tm = jnp.max(lt, axis=1, keepdims=True)
            m0 = tm if m0 is None else jnp.maximum(m0, tm)
            r1j = remote_copy(
                comm_ref.at[:, sub_cols(my, j)],
                comm_ref.at[:, sub_cols(my, j)],
                send_sems, recv_sems, j, right,
            )
            l1j = remote_copy(
                comm_ref.at[:, sub_cols(my, j)],
                comm_ref.at[:, sub_cols(my, j)],
                send_sems, recv_sems, NT + j, left,
            )
            r1j.start()
            l1j.start()
            hop1.append((r1j, l1j))
        lg_dma(my, 0).start()

        def s0_body(t, s):
            l = comm_ref[:, pl.ds(my * NL + t * TN, TN)].astype(jnp.float32)
            return s + jnp.sum(jnp.exp(l - m0), axis=1, keepdims=True)

        s0 = lax.fori_loop(
            0, NL // TN, s0_body, jnp.zeros((M, 1), jnp.float32)
        )
        stats_ref[0] = jnp.concatenate(
            [jnp.broadcast_to(m0, (M, 4)), jnp.broadcast_to(s0, (M, 4))],
            axis=1,
        )
        st_r1 = remote_copy(
            stats_ref.at[0], stats_ref.at[1], st_send, st_recv, 0, right
        )
        st_l1 = remote_copy(
            stats_ref.at[0], stats_ref.at[2], st_send, st_recv, 1, left
        )
        st_r1.start()
        st_l1.start()

        for r1j, _ in hop1:
            r1j.wait_recv()
        r2 = remote_copy(
            comm_ref.at[pl.ds(0, MH), cols(left)],
            comm_ref.at[pl.ds(0, MH), cols(left)],
            s2_send, s2_recv, 0, right,
        )
        r2.start()
        lg_dma(left, 1).start()
        st_r1.wait_recv()
        st_r2 = remote_copy(
            stats_ref.at[1], stats_ref.at[3], st_send, st_recv, 2, right
        )
        st_r2.start()
        for _, l1j in hop1:
            l1j.wait_recv()
        l2 = remote_copy(
            comm_ref.at[pl.ds(MH, MH), cols(right)],
            comm_ref.at[pl.ds(MH, MH), cols(right)],
            s2_send, s2_recv, 1, left,
        )
        l2.start()
        lg_dma(right, 2).start()
        for r1j, l1j in hop1:
            r1j.wait_send()
            l1j.wait_send()
        st_l1.wait_recv()

        r2.wait_recv()
        l2.wait_recv()
        r2.wait_send()
        l2.wait_send()
        st_r2.wait_recv()
        st_r1.wait_send()
        st_l1.wait_send()
        st_r2.wait_send()
        opp = lax.rem(my + 2, N_DEV)
        lg_dma(opp, 3).start()

        mc = [stats_ref[c, :, 0:1] for c in range(N_DEV)]
        sc = [stats_ref[c, :, 4:5] for c in range(N_DEV)]
        m_g = jnp.maximum(jnp.maximum(mc[0], mc[1]),
                          jnp.maximum(mc[2], mc[3]))
        s_g = sum(s * jnp.exp(m - m_g) for s, m in zip(sc, mc))
        m_ref[...] = jnp.broadcast_to(m_g, (M, 128))
        r_ref[...] = jnp.broadcast_to(1.0 / s_g, (M, 128))

        for origin, slot in ((my, 0), (left, 1), (right, 2), (opp, 3)):
            lg_dma(origin, slot).wait()

    return pl.pallas_call(
        body,
        out_shape=[
            jax.ShapeDtypeStruct((M, NG), jnp.bfloat16),
            jax.ShapeDtypeStruct((M, 128), jnp.float32),
            jax.ShapeDtypeStruct((M, 128), jnp.float32),
        ],
        in_specs=[
            pl.BlockSpec(memory_space=pltpu.VMEM),
            pl.BlockSpec(memory_space=pl.ANY),
        ],
        out_specs=[
            pl.BlockSpec(memory_space=pl.ANY),
            pl.BlockSpec(memory_space=pltpu.VMEM),
            pl.BlockSpec(memory_space=pltpu.VMEM),
        ],
        scratch_shapes=[
            pltpu.VMEM((M, NG), jnp.bfloat16),
            pltpu.VMEM((2, K, TW), jnp.float32),
            pltpu.VMEM((N_DEV, M, 8), jnp.float32),
            pltpu.SemaphoreType.DMA((2,)),
            pltpu.SemaphoreType.DMA((2 * NT,)),
            pltpu.SemaphoreType.DMA((2 * NT,)),
            pltpu.SemaphoreType.DMA((2,)),
            pltpu.SemaphoreType.DMA((2,)),
            pltpu.SemaphoreType.DMA((3,)),
            pltpu.SemaphoreType.DMA((3,)),
            pltpu.SemaphoreType.DMA((4,)),
        ],
        compiler_params=pltpu.CompilerParams(
            collective_id=0, vmem_limit_bytes=60 * 1024 * 1024
        ),
    )(x, W)


def _normalize(lg, m, r):
    def body(lg_ref, m_ref, r_ref, out_ref):
        m = m_ref[:, 0:1].astype(jnp.bfloat16)
        r = r_ref[:, 0:1].astype(jnp.bfloat16)
        out_ref[...] = jnp.exp(lg_ref[...] - m) * r

    TB = 4096
    return pl.pallas_call(
        body,
        grid=(NG // TB,),
        in_specs=[
            pl.BlockSpec((M, TB), lambda t: (0, t)),
            pl.BlockSpec((M, 128), lambda t: (0, 0)),
            pl.BlockSpec((M, 128), lambda t: (0, 0)),
        ],
        out_specs=pl.BlockSpec((M, TB), lambda t: (0, t)),
        out_shape=jax.ShapeDtypeStruct((M, NG), jnp.bfloat16),
    )(lg, m, r)


def kernel(x, W):
    lg, m, r = _gather_stats(x, W)
    return _normalize(lg, m, r)
